# baseline (speedup 1.0000x reference)
import jax
import jax.numpy as jnp
from jax import lax
from jax.experimental import pallas as pl
from jax.experimental.pallas import tpu as pltpu
from jax.experimental.pallas import tpu_sc as plsc

N = 10000
E = 320000
C = 128
NSC = 2
NT = 16
EPT = E // NT
K = 128
CH = 160
PAD = CH * K - EPT
NP = 2
CHP = CH // NP
NPAD = 10240
NH = NPAD // NSC
NPT = NH // NT
EB = 12800
RB = 2000
NEG = -1e30


def _dense_body(x_ref, w_ref, att2_ref, h_ref, ab_ref):
  h = jnp.dot(x_ref[...], w_ref[...], preferred_element_type=jnp.float32)
  h_ref[...] = h
  ab_ref[...] = jnp.dot(h, att2_ref[...], preferred_element_type=jnp.float32)


def _aedge_body(ea_ref, we_ref, aev_ref, out_ref):
  ve = jnp.sum(we_ref[...] * aev_ref[...], axis=1)
  out_ref[...] = jnp.sum(ea_ref[...] * ve[None, :], axis=1)[None, :]


def _final_body(agg_ref, s_ref, b_ref, out_ref):
  out_ref[...] = agg_ref[...] / (s_ref[...] + 1e-16) + b_ref[...]


def _sc_body(src_hbm, dst_hbm, ae_hbm, asrc_hbm, adst_hbm, h_hbm,
             s_out, agg_out,
             gsrc, gdst, w_v, asrc_v, adst_v, rows_a, rows_b, zb,
             agg_sh, s_sh, sem_a, sem_b):
  c = lax.axis_index("c")
  t = lax.axis_index("s")
  lo = c * NH
  z16 = jnp.zeros((16,), jnp.float32)

  @pl.loop(0, K)
  def _(r):
    for q in range(8):
      rows_a[r, pl.ds(q * 16, 16)] = z16

  @pl.loop(0, NPT // 16)
  def _(i):
    zb[pl.ds(i * 16, 16)] = z16

  pltpu.sync_copy(rows_a, agg_sh.at[pl.ds(t * NPT, 128)])
  pltpu.sync_copy(rows_a, agg_sh.at[pl.ds(t * NPT + 128, 128)])
  pltpu.sync_copy(rows_a.at[pl.ds(0, 64)], agg_sh.at[pl.ds(t * NPT + 256, 64)])
  pltpu.sync_copy(zb, s_sh.at[pl.ds(t * NPT, NPT)])

  pltpu.sync_copy(asrc_hbm, asrc_v)
  pltpu.sync_copy(adst_hbm, adst_v)
  plsc.subcore_barrier()

  def _gather(j, buf, sem):
    return pltpu.async_copy(
        h_hbm.at[plsc.Indices(gsrc.at[j], ignored_value=-1)], buf, sem)

  def _gwait(j, buf, sem):
    pltpu.make_async_copy(
        h_hbm.at[plsc.Indices(gsrc.at[j], ignored_value=-1)], buf, sem).wait()

  def _proc(j, cur):
    @pl.loop(0, 8)
    def _(g):
      wg = w_v[j, pl.ds(g * 16, 16)]
      for rr in range(16):
        wr = wg[rr]
        r = g * 16 + rr
        for q in range(8):
          sl = pl.ds(q * 16, 16)
          cur[r, sl] = cur[r, sl] * wr
    pltpu.sync_copy(
        cur, agg_sh.at[plsc.Indices(gdst.at[j], ignored_value=-1)], add=True)
    pltpu.sync_copy(
        w_v.at[j], s_sh.at[plsc.Indices(gdst.at[j], ignored_value=-1)],
        add=True)

  for p in range(NP):
    base = t * CH + p * CHP
    pltpu.sync_copy(src_hbm.at[pl.ds(base, CHP)], gsrc)
    pltpu.sync_copy(dst_hbm.at[pl.ds(base, CHP)], gdst)
    pltpu.sync_copy(ae_hbm.at[pl.ds(base, CHP)], w_v)

    @pl.loop(0, CHP)
    def _(j):
      for g in range(8):
        sl = pl.ds(g * 16, 16)
        s16 = gsrc[j, sl]
        d16 = gdst[j, sl]
        a = (plsc.load_gather(asrc_v, [s16])
             + plsc.load_gather(adst_v, [d16])
             + w_v[j, sl])
        a = jnp.where(a >= 0.0, a, a * 0.2)
        w_v[j, sl] = jnp.exp(a)
        dloc = d16 - lo
        owned = (dloc >= 0) & (dloc < NH)
        m1 = jnp.full((16,), -1, jnp.int32)
        gsrc[j, sl] = jnp.where(owned, s16, m1)
        gdst[j, sl] = jnp.where(owned, dloc, m1)

    _gather(0, rows_a, sem_a)

    @pl.loop(0, CHP, step=2)
    def _(j):
      _gwait(j, rows_a, sem_a)
      _gather(j + 1, rows_b, sem_b)
      _proc(j, rows_a)
      _gwait(j + 1, rows_b, sem_b)

      @pl.when(j + 2 < CHP)
      def _():
        _gather(j + 2, rows_a, sem_a)

      _proc(j + 1, rows_b)

  plsc.subcore_barrier()
  pltpu.sync_copy(agg_sh.at[pl.ds(t * NPT, NPT)],
                  agg_out.at[c, pl.ds(t * NPT, NPT)])

  @pl.when(t < 10)
  def _():
    pltpu.sync_copy(s_sh.at[pl.ds(t * 512, 512)],
                    s_out.at[pl.ds(c * NH + t * 512, 512)])


def _make_sc_call():
  mesh = plsc.VectorSubcoreMesh(core_axis_name="c", subcore_axis_name="s",
                                num_cores=NSC, num_subcores=NT)
  return pl.kernel(
      _sc_body,
      out_type=(
          jax.ShapeDtypeStruct((NPAD,), jnp.float32),
          jax.ShapeDtypeStruct((NSC, NH, C), jnp.float32),
      ),
      mesh=mesh,
      compiler_params=pltpu.CompilerParams(needs_layout_passes=False),
      scratch_types=[
          pltpu.VMEM((CHP, K), jnp.int32),
          pltpu.VMEM((CHP, K), jnp.int32),
          pltpu.VMEM((CHP, K), jnp.float32),
          pltpu.VMEM((NPAD,), jnp.float32),
          pltpu.VMEM((NPAD,), jnp.float32),
          pltpu.VMEM((K, C), jnp.float32),
          pltpu.VMEM((K, C), jnp.float32),
          pltpu.VMEM((NPT,), jnp.float32),
          pltpu.VMEM_SHARED((NH, C), jnp.float32),
          pltpu.VMEM_SHARED((NH,), jnp.float32),
          pltpu.SemaphoreType.DMA,
          pltpu.SemaphoreType.DMA,
      ],
  )


def _pad2(v, fill):
  return jnp.pad(v.reshape(NT, EPT), ((0, 0), (0, PAD)),
                 constant_values=fill).reshape(NT * CH, K)


@jax.jit
def kernel(x, edge_index, edge_attr, W, att_src, att_dst, W_edge, att_edge,
           bias):
  src = edge_index[0].astype(jnp.int32)
  dst = edge_index[1].astype(jnp.int32)
  aev = att_edge.reshape(1, C)
  att2 = jnp.concatenate(
      [att_src.reshape(C, 1), att_dst.reshape(C, 1)], axis=1)

  h, ab = pl.pallas_call(
      _dense_body,
      grid=(10,),
      in_specs=[
          pl.BlockSpec((N // 10, C), lambda i: (i, 0)),
          pl.BlockSpec((C, C), lambda i: (0, 0)),
          pl.BlockSpec((C, 2), lambda i: (0, 0)),
      ],
      out_specs=(
          pl.BlockSpec((N // 10, C), lambda i: (i, 0)),
          pl.BlockSpec((N // 10, 2), lambda i: (i, 0)),
      ),
      out_shape=(
          jax.ShapeDtypeStruct((N, C), jnp.float32),
          jax.ShapeDtypeStruct((N, 2), jnp.float32),
      ),
  )(x, W, att2)

  aeg = pl.pallas_call(
      _aedge_body,
      grid=(E // EB,),
      in_specs=[
          pl.BlockSpec((EB, 16), lambda i: (i, 0)),
          pl.BlockSpec((16, C), lambda i: (0, 0)),
          pl.BlockSpec((1, C), lambda i: (0, 0)),
      ],
      out_specs=pl.BlockSpec((1, EB), lambda i: (0, i)),
      out_shape=jax.ShapeDtypeStruct((1, E), jnp.float32),
  )(edge_attr, W_edge, aev)

  aedge = aeg.reshape(E)
  s_part, agg_part = _make_sc_call()(
      _pad2(src, 0), _pad2(dst, 0), _pad2(aedge, NEG),
      jnp.pad(ab[:, 0], (0, NPAD - N)),
      jnp.pad(ab[:, 1], (0, NPAD - N)), h)

  out = pl.pallas_call(
      _final_body,
      grid=(N // RB,),
      in_specs=[
          pl.BlockSpec((RB, C), lambda i: (i, 0)),
          pl.BlockSpec((RB, 1), lambda i: (i, 0)),
          pl.BlockSpec((1, C), lambda i: (0, 0)),
      ],
      out_specs=pl.BlockSpec((RB, C), lambda i: (i, 0)),
      out_shape=jax.ShapeDtypeStruct((N, C), jnp.float32),
  )(agg_part.reshape(NPAD, C), s_part.reshape(NPAD, 1), bias.reshape(1, C))
  return out


# speedup vs baseline: 1.0047x; 1.0047x over previous
import jax
import jax.numpy as jnp
from jax import lax
from jax.experimental import pallas as pl
from jax.experimental.pallas import tpu as pltpu
from jax.experimental.pallas import tpu_sc as plsc

N = 10000
E = 320000
C = 128
NSC = 2
NT = 16
EPT = E // NT
K = 128
CH = 160
PAD = CH * K - EPT
NP = 2
CHP = CH // NP
NPAD = 10240
NH = NPAD // NSC
NPT = NH // NT
EB = 12800
RB = 2000
NEG = -1e30


def _dense_body(x_ref, w_ref, att2_ref, ea_ref, we_ref, aev_ref,
                h_ref, ab_ref, ae_ref):
  h = jnp.dot(x_ref[...], w_ref[...], preferred_element_type=jnp.float32)
  h_ref[...] = h
  ab_ref[...] = jnp.dot(h, att2_ref[...], preferred_element_type=jnp.float32)
  ve = jnp.sum(we_ref[...] * aev_ref[...], axis=1)
  ae_ref[...] = jnp.sum(ea_ref[...] * ve[None, :], axis=1)[None, :]


def _sc_body(src_hbm, dst_hbm, ae_hbm, asrc_hbm, adst_hbm, h_hbm, bias_hbm,
             out_hbm,
             gsrc, gdst, w_v, asrc_v, adst_v, rows_a, rows_b, zb, bias_v,
             agg_sh, s_sh, sem_a, sem_b):
  c = lax.axis_index("c")
  t = lax.axis_index("s")
  lo = c * NH
  z16 = jnp.zeros((16,), jnp.float32)

  @pl.loop(0, K)
  def _(r):
    for q in range(8):
      rows_a[r, pl.ds(q * 16, 16)] = z16

  @pl.loop(0, NPT // 16)
  def _(i):
    zb[pl.ds(i * 16, 16)] = z16

  pltpu.sync_copy(rows_a, agg_sh.at[pl.ds(t * NPT, 128)])
  pltpu.sync_copy(rows_a, agg_sh.at[pl.ds(t * NPT + 128, 128)])
  pltpu.sync_copy(rows_a.at[pl.ds(0, 64)], agg_sh.at[pl.ds(t * NPT + 256, 64)])
  pltpu.sync_copy(zb, s_sh.at[pl.ds(t * NPT, NPT)])

  pltpu.sync_copy(asrc_hbm, asrc_v)
  pltpu.sync_copy(adst_hbm, adst_v)
  plsc.subcore_barrier()

  def _gather(j, buf, sem):
    return pltpu.async_copy(
        h_hbm.at[plsc.Indices(gsrc.at[j], ignored_value=-1)], buf, sem)

  def _gwait(j, buf, sem):
    pltpu.make_async_copy(
        h_hbm.at[plsc.Indices(gsrc.at[j], ignored_value=-1)], buf, sem).wait()

  def _proc(j, cur):
    @pl.loop(0, 8)
    def _(g):
      wg = w_v[j, pl.ds(g * 16, 16)]
      for rr in range(16):
        wr = wg[rr]
        r = g * 16 + rr
        for q in range(8):
          sl = pl.ds(q * 16, 16)
          cur[r, sl] = cur[r, sl] * wr
    pltpu.sync_copy(
        cur, agg_sh.at[plsc.Indices(gdst.at[j], ignored_value=-1)], add=True)
    pltpu.sync_copy(
        w_v.at[j], s_sh.at[plsc.Indices(gdst.at[j], ignored_value=-1)],
        add=True)

  for p in range(NP):
    base = t * CH + p * CHP
    pltpu.sync_copy(src_hbm.at[pl.ds(base, CHP)], gsrc)
    pltpu.sync_copy(dst_hbm.at[pl.ds(base, CHP)], gdst)
    pltpu.sync_copy(ae_hbm.at[pl.ds(base, CHP)], w_v)

    @pl.loop(0, CHP)
    def _(j):
      for g in range(8):
        sl = pl.ds(g * 16, 16)
        s16 = gsrc[j, sl]
        d16 = gdst[j, sl]
        a = (plsc.load_gather(asrc_v, [s16])
             + plsc.load_gather(adst_v, [d16])
             + w_v[j, sl])
        a = jnp.where(a >= 0.0, a, a * 0.2)
        w_v[j, sl] = jnp.exp(a)
        dloc = d16 - lo
        owned = (dloc >= 0) & (dloc < NH)
        m1 = jnp.full((16,), -1, jnp.int32)
        gsrc[j, sl] = jnp.where(owned, s16, m1)
        gdst[j, sl] = jnp.where(owned, dloc, m1)

    _gather(0, rows_a, sem_a)

    @pl.loop(0, CHP, step=2)
    def _(j):
      _gwait(j, rows_a, sem_a)
      _gather(j + 1, rows_b, sem_b)
      _proc(j, rows_a)
      _gwait(j + 1, rows_b, sem_b)

      @pl.when(j + 2 < CHP)
      def _():
        _gather(j + 2, rows_a, sem_a)

      _proc(j + 1, rows_b)

  plsc.subcore_barrier()

  pltpu.sync_copy(s_sh.at[pl.ds(t * NPT, NPT)], zb)
  pltpu.sync_copy(bias_hbm, bias_v)
  start = c * NH + t * NPT

  def _finalize(bc, nrows):
    pltpu.sync_copy(agg_sh.at[pl.ds(t * NPT + bc * 128, 128)], rows_a)

    @pl.loop(0, 8)
    def _(g):
      sv = zb[pl.ds(bc * 128 + g * 16, 16)]
      inv = 1.0 / (sv + 1e-16)
      for rr in range(16):
        ivr = inv[rr]
        r = g * 16 + rr
        for q in range(8):
          sl = pl.ds(q * 16, 16)
          rows_a[r, sl] = rows_a[r, sl] * ivr + bias_v[pl.ds(q * 16, 16)]
    if nrows == 128:
      pltpu.sync_copy(rows_a, out_hbm.at[pl.ds(start + bc * 128, 128)])
    else:
      pltpu.sync_copy(rows_a.at[pl.ds(0, nrows)],
                      out_hbm.at[pl.ds(start + bc * 128, nrows)])

  is_last = start >= N - NPT + 128

  @pl.when(jnp.logical_not(is_last))
  def _():
    _finalize(0, 128)
    _finalize(1, 128)
    _finalize(2, 64)

  @pl.when(is_last)
  def _():
    _finalize(0, 80)


def _make_sc_call():
  mesh = plsc.VectorSubcoreMesh(core_axis_name="c", subcore_axis_name="s",
                                num_cores=NSC, num_subcores=NT)
  return pl.kernel(
      _sc_body,
      out_type=jax.ShapeDtypeStruct((N, C), jnp.float32),
      mesh=mesh,
      compiler_params=pltpu.CompilerParams(needs_layout_passes=False),
      scratch_types=[
          pltpu.VMEM((CHP, K), jnp.int32),
          pltpu.VMEM((CHP, K), jnp.int32),
          pltpu.VMEM((CHP, K), jnp.float32),
          pltpu.VMEM((NPAD,), jnp.float32),
          pltpu.VMEM((NPAD,), jnp.float32),
          pltpu.VMEM((K, C), jnp.float32),
          pltpu.VMEM((K, C), jnp.float32),
          pltpu.VMEM((NPT,), jnp.float32),
          pltpu.VMEM((C,), jnp.float32),
          pltpu.VMEM_SHARED((NH, C), jnp.float32),
          pltpu.VMEM_SHARED((NH,), jnp.float32),
          pltpu.SemaphoreType.DMA,
          pltpu.SemaphoreType.DMA,
      ],
  )


def _pad2(v, fill):
  return jnp.pad(v.reshape(NT, EPT), ((0, 0), (0, PAD)),
                 constant_values=fill).reshape(NT * CH, K)


@jax.jit
def kernel(x, edge_index, edge_attr, W, att_src, att_dst, W_edge, att_edge,
           bias):
  src = edge_index[0].astype(jnp.int32)
  dst = edge_index[1].astype(jnp.int32)
  aev = att_edge.reshape(1, C)
  att2 = jnp.concatenate(
      [att_src.reshape(C, 1), att_dst.reshape(C, 1)], axis=1)

  h, ab, aeg = pl.pallas_call(
      _dense_body,
      grid=(10,),
      in_specs=[
          pl.BlockSpec((N // 10, C), lambda i: (i, 0)),
          pl.BlockSpec((C, C), lambda i: (0, 0)),
          pl.BlockSpec((C, 2), lambda i: (0, 0)),
          pl.BlockSpec((E // 10, 16), lambda i: (i, 0)),
          pl.BlockSpec((16, C), lambda i: (0, 0)),
          pl.BlockSpec((1, C), lambda i: (0, 0)),
      ],
      out_specs=(
          pl.BlockSpec((N // 10, C), lambda i: (i, 0)),
          pl.BlockSpec((N // 10, 2), lambda i: (i, 0)),
          pl.BlockSpec((1, E // 10), lambda i: (0, i)),
      ),
      out_shape=(
          jax.ShapeDtypeStruct((N, C), jnp.float32),
          jax.ShapeDtypeStruct((N, 2), jnp.float32),
          jax.ShapeDtypeStruct((1, E), jnp.float32),
      ),
  )(x, W, att2, edge_attr, W_edge, aev)

  aedge = aeg.reshape(E)
  out = _make_sc_call()(
      _pad2(src, 0), _pad2(dst, 0), _pad2(aedge, NEG),
      jnp.pad(ab[:, 0], (0, NPAD - N)),
      jnp.pad(ab[:, 1], (0, NPAD - N)), h, bias)
  return out


# speedup vs baseline: 1.0527x; 1.0478x over previous
import jax
import jax.numpy as jnp
from jax import lax
from jax.experimental import pallas as pl
from jax.experimental.pallas import tpu as pltpu
from jax.experimental.pallas import tpu_sc as plsc

N = 10000
E = 320000
C = 128
NSC = 2
NT = 16
EPT = E // NT
K = 128
CH = 160
PAD = CH * K - EPT
NP = 2
CHP = CH // NP
NPAD = 10240
NH = NPAD // NSC
NPT = NH // NT
EB = 12800
RB = 2000
NEG = -1e30


def _dense_body(x_ref, w_ref, att2_ref, ea_ref, we_ref, aev_ref,
                h_ref, ab_ref, ae_ref):
  h = jnp.dot(x_ref[...], w_ref[...], preferred_element_type=jnp.float32)
  h_ref[...] = h
  ab_ref[...] = jnp.dot(h, att2_ref[...], preferred_element_type=jnp.float32)
  ve = jnp.sum(we_ref[...] * aev_ref[...], axis=1)
  ae_ref[...] = jnp.sum(ea_ref[...] * ve[None, :], axis=1)[None, :]


def _sc_body(src_hbm, dst_hbm, ae_hbm, asrc_hbm, adst_hbm, h_hbm, bias_hbm,
             out_hbm,
             gsrc, gdst, w_v, asrc_v, adst_v, rows_a, rows_b, zb, bias_v,
             agg_sh, s_sh, sem_a, sem_b):
  c = lax.axis_index("c")
  t = lax.axis_index("s")
  lo = c * NH
  z16 = jnp.zeros((16,), jnp.float32)

  @pl.loop(0, K)
  def _(r):
    for q in range(8):
      rows_a[r, pl.ds(q * 16, 16)] = z16

  @pl.loop(0, NPT // 16)
  def _(i):
    zb[pl.ds(i * 16, 16)] = z16

  pltpu.sync_copy(rows_a, agg_sh.at[pl.ds(t * NPT, 128)])
  pltpu.sync_copy(rows_a, agg_sh.at[pl.ds(t * NPT + 128, 128)])
  pltpu.sync_copy(rows_a.at[pl.ds(0, 64)], agg_sh.at[pl.ds(t * NPT + 256, 64)])
  pltpu.sync_copy(zb, s_sh.at[pl.ds(t * NPT, NPT)])

  pltpu.sync_copy(asrc_hbm, asrc_v)
  pltpu.sync_copy(adst_hbm, adst_v)
  plsc.subcore_barrier()

  iota16 = lax.iota(jnp.int32, 16)
  m1 = jnp.full((16,), -1, jnp.int32)
  zf = jnp.zeros((16,), jnp.float32)

  def _gather(j, buf, sem):
    return pltpu.async_copy(
        h_hbm.at[plsc.Indices(gsrc.at[pl.ds(j * K, K)], ignored_value=-1)],
        buf, sem)

  def _gwait(j, buf, sem):
    pltpu.make_async_copy(
        h_hbm.at[plsc.Indices(gsrc.at[pl.ds(j * K, K)], ignored_value=-1)],
        buf, sem).wait()

  def _proc(j, cur):
    @pl.loop(0, 8)
    def _(g):
      off = pl.ds(j * K + g * 16, 16)
      wg = w_v[off]
      dg = gdst[off]
      for rr in range(16):
        wr = wg[rr]
        r = g * 16 + rr
        for q in range(8):
          sl = pl.ds(q * 16, 16)
          cur[r, sl] = cur[r, sl] * wr
      pltpu.sync_copy(cur.at[pl.ds(g * 16, 16)],
                      agg_sh.at[plsc.Indices(dg, ignored_value=-1)], add=True)
      pltpu.sync_copy(w_v.at[off],
                      s_sh.at[plsc.Indices(dg, ignored_value=-1)], add=True)

  for p in range(NP):
    base = (t * CH + p * CHP) * K
    epp = CHP * K
    pltpu.sync_copy(src_hbm.at[pl.ds(base, epp)], gsrc.at[pl.ds(0, epp)])
    pltpu.sync_copy(dst_hbm.at[pl.ds(base, epp)], gdst.at[pl.ds(0, epp)])
    pltpu.sync_copy(ae_hbm.at[pl.ds(base, epp)], w_v.at[pl.ds(0, epp)])

    @pl.loop(0, epp // 16, init_carry=0)
    def _compact(i, cnt):
      sl = pl.ds(i * 16, 16)
      s16 = gsrc[sl]
      d16 = gdst[sl]
      a = (plsc.load_gather(asrc_v, [s16])
           + plsc.load_gather(adst_v, [d16])
           + w_v[sl])
      a = jnp.where(a >= 0.0, a, a * 0.2)
      w16 = jnp.exp(a)
      dloc = d16 - lo
      owned = (dloc >= 0) & (dloc < NH)
      plsc.store_compressed(gsrc.at[pl.ds(cnt, 16)], s16, mask=owned)
      plsc.store_compressed(gdst.at[pl.ds(cnt, 16)], dloc, mask=owned)
      plsc.store_compressed(w_v.at[pl.ds(cnt, 16)], w16, mask=owned)
      return cnt + plsc.all_reduce_population_count(owned)[0]

    cnt = _compact
    for q in range(8):
      idx = cnt + q * 16 + iota16
      plsc.store_scatter(gsrc, [idx], m1)
      plsc.store_scatter(gdst, [idx], m1)
      plsc.store_scatter(w_v, [idx], zf)

    ncg = jnp.maximum((cnt + K - 1) // K, 1)
    nce = (ncg // 2) * 2

    _gather(0, rows_a, sem_a)

    @pl.loop(0, nce, step=2)
    def _(j):
      _gwait(j, rows_a, sem_a)
      _gather(j + 1, rows_b, sem_b)
      _proc(j, rows_a)
      _gwait(j + 1, rows_b, sem_b)

      @pl.when(j + 2 < ncg)
      def _():
        _gather(j + 2, rows_a, sem_a)

      _proc(j + 1, rows_b)

    @pl.when(ncg != nce)
    def _():
      _gwait(ncg - 1, rows_a, sem_a)
      _proc(ncg - 1, rows_a)

  plsc.subcore_barrier()

  pltpu.sync_copy(s_sh.at[pl.ds(t * NPT, NPT)], zb)
  pltpu.sync_copy(bias_hbm, bias_v)
  start = c * NH + t * NPT

  def _finalize(bc, nrows):
    pltpu.sync_copy(agg_sh.at[pl.ds(t * NPT + bc * 128, 128)], rows_a)

    @pl.loop(0, 8)
    def _(g):
      sv = zb[pl.ds(bc * 128 + g * 16, 16)]
      inv = 1.0 / (sv + 1e-16)
      for rr in range(16):
        ivr = inv[rr]
        r = g * 16 + rr
        for q in range(8):
          sl = pl.ds(q * 16, 16)
          rows_a[r, sl] = rows_a[r, sl] * ivr + bias_v[pl.ds(q * 16, 16)]
    if nrows == 128:
      pltpu.sync_copy(rows_a, out_hbm.at[pl.ds(start + bc * 128, 128)])
    else:
      pltpu.sync_copy(rows_a.at[pl.ds(0, nrows)],
                      out_hbm.at[pl.ds(start + bc * 128, nrows)])

  is_last = start >= N - NPT + 128

  @pl.when(jnp.logical_not(is_last))
  def _():
    _finalize(0, 128)
    _finalize(1, 128)
    _finalize(2, 64)

  @pl.when(is_last)
  def _():
    _finalize(0, 80)


def _make_sc_call():
  mesh = plsc.VectorSubcoreMesh(core_axis_name="c", subcore_axis_name="s",
                                num_cores=NSC, num_subcores=NT)
  return pl.kernel(
      _sc_body,
      out_type=jax.ShapeDtypeStruct((N, C), jnp.float32),
      mesh=mesh,
      compiler_params=pltpu.CompilerParams(needs_layout_passes=False),
      scratch_types=[
          pltpu.VMEM((CHP * K + K,), jnp.int32),
          pltpu.VMEM((CHP * K + K,), jnp.int32),
          pltpu.VMEM((CHP * K + K,), jnp.float32),
          pltpu.VMEM((NPAD,), jnp.float32),
          pltpu.VMEM((NPAD,), jnp.float32),
          pltpu.VMEM((K, C), jnp.float32),
          pltpu.VMEM((K, C), jnp.float32),
          pltpu.VMEM((NPT,), jnp.float32),
          pltpu.VMEM((C,), jnp.float32),
          pltpu.VMEM_SHARED((NH, C), jnp.float32),
          pltpu.VMEM_SHARED((NH,), jnp.float32),
          pltpu.SemaphoreType.DMA,
          pltpu.SemaphoreType.DMA,
      ],
  )


def _pad2(v, fill):
  return jnp.pad(v.reshape(NT, EPT), ((0, 0), (0, PAD)),
                 constant_values=fill).reshape(NT * CH, K)


@jax.jit
def kernel(x, edge_index, edge_attr, W, att_src, att_dst, W_edge, att_edge,
           bias):
  src = edge_index[0].astype(jnp.int32)
  dst = edge_index[1].astype(jnp.int32)
  aev = att_edge.reshape(1, C)
  att2 = jnp.concatenate(
      [att_src.reshape(C, 1), att_dst.reshape(C, 1)], axis=1)

  h, ab, aeg = pl.pallas_call(
      _dense_body,
      grid=(10,),
      in_specs=[
          pl.BlockSpec((N // 10, C), lambda i: (i, 0)),
          pl.BlockSpec((C, C), lambda i: (0, 0)),
          pl.BlockSpec((C, 2), lambda i: (0, 0)),
          pl.BlockSpec((E // 10, 16), lambda i: (i, 0)),
          pl.BlockSpec((16, C), lambda i: (0, 0)),
          pl.BlockSpec((1, C), lambda i: (0, 0)),
      ],
      out_specs=(
          pl.BlockSpec((N // 10, C), lambda i: (i, 0)),
          pl.BlockSpec((N // 10, 2), lambda i: (i, 0)),
          pl.BlockSpec((1, E // 10), lambda i: (0, i)),
      ),
      out_shape=(
          jax.ShapeDtypeStruct((N, C), jnp.float32),
          jax.ShapeDtypeStruct((N, 2), jnp.float32),
          jax.ShapeDtypeStruct((1, E), jnp.float32),
      ),
  )(x, W, att2, edge_attr, W_edge, aev)

  aedge = aeg.reshape(E)
  out = _make_sc_call()(
      _pad2(src, 0).reshape(NT * CH * K), _pad2(dst, 0).reshape(NT * CH * K),
      _pad2(aedge, NEG).reshape(NT * CH * K),
      jnp.pad(ab[:, 0], (0, NPAD - N)),
      jnp.pad(ab[:, 1], (0, NPAD - N)), h, bias)
  return out
